# baseline (speedup 1.0000x reference)
import functools

import jax
import jax.numpy as jnp
from jax import lax
from jax.experimental import pallas as pl
from jax.experimental.pallas import tpu as pltpu
from jax.experimental.pallas import tpu_sc as plsc

N_VOCAB = 1000000
N_EMBED = 64
N_TOKEN = 50
BATCH = 4096

NC, NS = 2, 16
NW = NC * NS
BPW = BATCH // NW
LANES = 16
KQ = N_EMBED // LANES


def _sc_body(tok_hbm, tab_hbm, pos_hbm, out_hbm,
             idx_v, pos_v, buf0, buf1, ob0, ob1,
             gsem0, gsem1, osem0, osem1):
    wid = lax.axis_index("s") * NC + lax.axis_index("c")
    b0 = wid * BPW

    pltpu.sync_copy(tok_hbm.at[:, pl.ds(b0, BPW)], idx_v)
    pltpu.sync_copy(pos_hbm, pos_v)

    iota16 = lax.iota(jnp.int32, LANES)
    rowsq = [iota16 + q * LANES for q in range(KQ)]
    bufs = (buf0, buf1)
    obs = (ob0, ob1)
    gsems = (gsem0, gsem1)
    osems = (osem0, osem1)
    ghandles = [None, None]
    ohandles = [None, None]

    def start_gather(t):
        b = t % 2
        ghandles[b] = pltpu.async_copy(
            tab_hbm.at[idx_v.at[t]], bufs[b], gsems[b])

    def compute(t, buf, ob):
        pq = [pos_v[t, pl.ds(q * LANES, LANES)] for q in range(KQ)]

        def b_body(b, c):
            bfull = jnp.full((LANES,), b, jnp.int32)
            for q in range(KQ):
                val = buf[b, pl.ds(q * LANES, LANES)] + pq[q]
                plsc.store_scatter(ob, [rowsq[q], bfull], val)
            return c

        lax.fori_loop(0, BPW, b_body, 0)

    start_gather(0)
    start_gather(1)
    for t in range(N_TOKEN):
        b = t % 2
        ghandles[b].wait()
        if ohandles[b] is not None:
            ohandles[b].wait()
            ohandles[b] = None
        compute(t, bufs[b], obs[b])
        ohandles[b] = pltpu.async_copy(
            obs[b], out_hbm.at[t, :, pl.ds(b0, BPW)], osems[b])
        if t + 2 < N_TOKEN:
            start_gather(t + 2)
    for b in range(2):
        if ohandles[b] is not None:
            ohandles[b].wait()


@jax.jit
def _run(tokens_t, table_pad, pos):
    sc = functools.partial(
        pl.kernel,
        mesh=plsc.VectorSubcoreMesh(core_axis_name="c", subcore_axis_name="s"),
        compiler_params=pltpu.CompilerParams(
            use_tc_tiling_on_sc=True, needs_layout_passes=False),
        out_type=jax.ShapeDtypeStruct((N_TOKEN, N_EMBED, BATCH), jnp.float32),
        scratch_types=[
            pltpu.VMEM((N_TOKEN, BPW), jnp.int32),
            pltpu.VMEM((N_TOKEN, N_EMBED), jnp.float32),
            pltpu.VMEM((BPW, 128), jnp.float32),
            pltpu.VMEM((BPW, 128), jnp.float32),
            pltpu.VMEM((N_EMBED, BPW), jnp.float32),
            pltpu.VMEM((N_EMBED, BPW), jnp.float32),
            pltpu.SemaphoreType.DMA,
            pltpu.SemaphoreType.DMA,
            pltpu.SemaphoreType.DMA,
            pltpu.SemaphoreType.DMA,
        ],
    )
    return sc(_sc_body)(tokens_t, table_pad, pos)


def kernel(tokens, token_embedding, position_embedding):
    tokens_t = jnp.asarray(tokens, jnp.int32).T
    table_pad = jnp.pad(token_embedding, ((0, 0), (0, N_EMBED)))
    out_t = _run(tokens_t, table_pad, position_embedding)
    return jnp.transpose(out_t, (2, 0, 1))


# speedup vs baseline: 1.0501x; 1.0501x over previous
import functools

import jax
import jax.numpy as jnp
from jax import lax
from jax.experimental import pallas as pl
from jax.experimental.pallas import tpu as pltpu
from jax.experimental.pallas import tpu_sc as plsc

N_VOCAB = 1000000
N_EMBED = 64
N_TOKEN = 50
BATCH = 4096

ROWS = BATCH * N_TOKEN
NC, NS = 2, 16
NW = NC * NS
BPW = BATCH // NW
LANES = 16


def _sc_body(tok_hbm, tab_hbm, inter_hbm, par_hbm,
             idx_v, pidx_v, par_v, buf_v, gsem0, gsem1, ssem0, ssem1, psem):
    wid = lax.axis_index("s") * NC + lax.axis_index("c")
    b0 = wid * BPW

    pltpu.sync_copy(tok_hbm.at[:, pl.ds(b0, BPW)], idx_v)

    def prep_row(t, carry):
        for j in range(BPW // LANES):
            sl = pl.ds(j * LANES, LANES)
            v = idx_v[t, sl]
            pidx_v[t, sl] = lax.bitwise_or(
                lax.shift_left(lax.shift_right_logical(v, 11), 10),
                lax.bitwise_and(v, 1023))
            p = lax.bitwise_and(lax.shift_right_logical(v, 10), 1)
            for r in range(8):
                par_v[t, r, sl] = p
        return carry

    lax.fori_loop(0, N_TOKEN, prep_row, 0)
    par_handle = pltpu.async_copy(
        par_v, par_hbm.at[:, :, pl.ds(b0, BPW)], psem)

    gsems = (gsem0, gsem1)
    ssems = (ssem0, ssem1)
    gather_handles = [None, None]
    store_handles = [None, None]

    def start_gather(t, b):
        if store_handles[b] is not None:
            store_handles[b].wait()
            store_handles[b] = None
        gather_handles[b] = pltpu.async_copy(
            tab_hbm.at[pidx_v.at[t]], buf_v.at[b], gsems[b])

    start_gather(0, 0)
    for t in range(N_TOKEN):
        b = t % 2
        if t + 1 < N_TOKEN:
            start_gather(t + 1, 1 - b)
        gather_handles[b].wait()
        store_handles[b] = pltpu.async_copy(
            buf_v.at[b],
            inter_hbm.at[pl.ds(t * BATCH + b0, BPW)],
            ssems[b])
    for b in range(2):
        if store_handles[b] is not None:
            store_handles[b].wait()
    par_handle.wait()


_GB = 2048
_NPAIR_BLOCKS = (N_VOCAB + _GB - 1) // _GB
_NPAIR = _NPAIR_BLOCKS * (_GB // 2)


def _pack_body(tp_ref, o_ref):
    x = tp_ref[...]
    y = jnp.transpose(x)
    o_ref[...] = jnp.concatenate([y[: _GB // 2], y[_GB // 2:]], axis=1)


def _pack_table(table_t):
    return pl.pallas_call(
        _pack_body,
        grid=(_NPAIR_BLOCKS,),
        in_specs=[pl.BlockSpec((N_EMBED, _GB), lambda j: (0, j))],
        out_specs=pl.BlockSpec((_GB // 2, 128), lambda j: (j, 0)),
        out_shape=jax.ShapeDtypeStruct((_NPAIR, 128), jnp.float32),
    )(table_t)


@jax.jit
def _run(tokens_t, table_t, pos_x):
    table_pairs = _pack_table(table_t)
    sc_gather = functools.partial(
        pl.kernel,
        mesh=plsc.VectorSubcoreMesh(core_axis_name="c", subcore_axis_name="s"),
        compiler_params=pltpu.CompilerParams(use_tc_tiling_on_sc=True),
        out_type=(
            jax.ShapeDtypeStruct((ROWS, 128), jnp.float32),
            jax.ShapeDtypeStruct((N_TOKEN, 8, BATCH), jnp.int32),
        ),
        scratch_types=[
            pltpu.VMEM((N_TOKEN, BPW), jnp.int32),
            pltpu.VMEM((N_TOKEN, BPW), jnp.int32),
            pltpu.VMEM((N_TOKEN, 8, BPW), jnp.int32),
            pltpu.VMEM((2, BPW, 128), jnp.float32),
            pltpu.SemaphoreType.DMA,
            pltpu.SemaphoreType.DMA,
            pltpu.SemaphoreType.DMA,
            pltpu.SemaphoreType.DMA,
            pltpu.SemaphoreType.DMA,
        ],
    )
    inter, par8 = sc_gather(_sc_body)(tokens_t, table_pairs)

    bb = 512
    nj = BATCH // bb

    def _tc_body(x_ref, par_ref, pos_ref, eye_ref, o_ref):
        xt = lax.dot_general(eye_ref[...], x_ref[...],
                             (((1,), (1,)), ((), ())),
                             preferred_element_type=jnp.float32)
        lo = xt[:N_EMBED, :]
        hi = xt[N_EMBED:, :]
        par = par_ref[0, 0:1, :]
        sel = jnp.where(par == 1, hi, lo)
        posv = pos_ref[0][:, 0:1]
        o_ref[0] = sel + posv

    eye = jnp.eye(128, dtype=jnp.float32)
    out_t = pl.pallas_call(
        _tc_body,
        grid=(N_TOKEN, nj),
        in_specs=[
            pl.BlockSpec((bb, 128), lambda t, j: (t * nj + j, 0)),
            pl.BlockSpec((1, 8, bb), lambda t, j: (t, 0, j)),
            pl.BlockSpec((1, N_EMBED, 128), lambda t, j: (t, 0, 0)),
            pl.BlockSpec((128, 128), lambda t, j: (0, 0)),
        ],
        out_specs=pl.BlockSpec((1, N_EMBED, bb), lambda t, j: (t, 0, j)),
        out_shape=jax.ShapeDtypeStruct((N_TOKEN, N_EMBED, BATCH), jnp.float32),
    )(inter, par8, pos_x, eye)
    return out_t


def kernel(tokens, token_embedding, position_embedding):
    tokens_t = jnp.asarray(tokens, jnp.int32).T
    table_t = token_embedding.T
    pos_x = jnp.broadcast_to(
        position_embedding[:, :, None], (N_TOKEN, N_EMBED, 128))
    out_t = _run(tokens_t, table_t, pos_x)
    return jnp.transpose(out_t, (2, 0, 1))


# speedup vs baseline: 1.9897x; 1.8947x over previous
import functools

import jax
import jax.numpy as jnp
from jax import lax
from jax.experimental import pallas as pl
from jax.experimental.pallas import tpu as pltpu
from jax.experimental.pallas import tpu_sc as plsc

N_VOCAB = 1000000
N_EMBED = 64
N_TOKEN = 50
BATCH = 4096

ROWS = BATCH * N_TOKEN
NC, NS = 2, 16
NW = NC * NS
BPW = BATCH // NW
LANES = 16

_GB = 8192
_GBLOG = 13
_HB = _GB // 2
_HBLOG = 12


def _sc_body(tok_hbm, tab_hbm, inter_hbm, par_hbm,
             idx_v, pidx_v, par_v, buf_v, gsem0, gsem1, ssem0, ssem1, psem):
    wid = lax.axis_index("s") * NC + lax.axis_index("c")
    b0 = wid * BPW

    pltpu.sync_copy(tok_hbm.at[:, pl.ds(b0, BPW)], idx_v)

    def prep_row(t, carry):
        for j in range(BPW // LANES):
            sl = pl.ds(j * LANES, LANES)
            v = idx_v[t, sl]
            pidx_v[t, sl] = lax.bitwise_or(
                lax.shift_left(lax.shift_right_logical(v, _GBLOG), _HBLOG),
                lax.bitwise_and(v, _HB - 1))
            p = lax.bitwise_and(lax.shift_right_logical(v, _HBLOG), 1)
            for r in range(8):
                par_v[t, r, sl] = p
        return carry

    lax.fori_loop(0, N_TOKEN, prep_row, 0)
    par_handle = pltpu.async_copy(
        par_v, par_hbm.at[:, :, pl.ds(b0, BPW)], psem)

    gsems = (gsem0, gsem1)
    ssems = (ssem0, ssem1)
    gather_handles = [None, None]
    store_handles = [None, None]

    def start_gather(t, b):
        if store_handles[b] is not None:
            store_handles[b].wait()
            store_handles[b] = None
        gather_handles[b] = pltpu.async_copy(
            tab_hbm.at[pidx_v.at[t]], buf_v.at[b], gsems[b])

    start_gather(0, 0)
    for t in range(N_TOKEN):
        b = t % 2
        if t + 1 < N_TOKEN:
            start_gather(t + 1, 1 - b)
        gather_handles[b].wait()
        store_handles[b] = pltpu.async_copy(
            buf_v.at[b],
            inter_hbm.at[pl.ds(t * BATCH + b0, BPW)],
            ssems[b])
    for b in range(2):
        if store_handles[b] is not None:
            store_handles[b].wait()
    par_handle.wait()


_NPAIR_BLOCKS = (N_VOCAB + _GB - 1) // _GB
_NPAIR = _NPAIR_BLOCKS * (_GB // 2)


def _pack_body(tp_ref, o_ref):
    x = tp_ref[...]
    y = jnp.transpose(x)
    o_ref[...] = jnp.concatenate([y[: _GB // 2], y[_GB // 2:]], axis=1)


def _pack_table(table_t):
    return pl.pallas_call(
        _pack_body,
        grid=(_NPAIR_BLOCKS,),
        in_specs=[pl.BlockSpec((N_EMBED, _GB), lambda j: (0, j))],
        out_specs=pl.BlockSpec((_GB // 2, 128), lambda j: (j, 0)),
        out_shape=jax.ShapeDtypeStruct((_NPAIR, 128), jnp.float32),
    )(table_t)


@jax.jit
def _run(tokens_t, table_t, pos_x):
    table_pairs = _pack_table(table_t)
    sc_gather = functools.partial(
        pl.kernel,
        mesh=plsc.VectorSubcoreMesh(core_axis_name="c", subcore_axis_name="s"),
        compiler_params=pltpu.CompilerParams(use_tc_tiling_on_sc=True),
        out_type=(
            jax.ShapeDtypeStruct((ROWS, 128), jnp.float32),
            jax.ShapeDtypeStruct((N_TOKEN, 8, BATCH), jnp.int32),
        ),
        scratch_types=[
            pltpu.VMEM((N_TOKEN, BPW), jnp.int32),
            pltpu.VMEM((N_TOKEN, BPW), jnp.int32),
            pltpu.VMEM((N_TOKEN, 8, BPW), jnp.int32),
            pltpu.VMEM((2, BPW, 128), jnp.float32),
            pltpu.SemaphoreType.DMA,
            pltpu.SemaphoreType.DMA,
            pltpu.SemaphoreType.DMA,
            pltpu.SemaphoreType.DMA,
            pltpu.SemaphoreType.DMA,
        ],
    )
    inter, par8 = sc_gather(_sc_body)(tokens_t, table_pairs)

    bb = BATCH

    def _tc_body(x_ref, par_ref, pos_ref, eye_ref, o_ref):
        xt = lax.dot_general(eye_ref[...], x_ref[...],
                             (((1,), (1,)), ((), ())),
                             preferred_element_type=jnp.float32)
        lo = xt[:N_EMBED, :]
        hi = xt[N_EMBED:, :]
        par = par_ref[0, 0:1, :]
        sel = jnp.where(par == 1, hi, lo)
        posv = pos_ref[0][:, 0:1]
        o_ref[0] = sel + posv

    eye = jnp.eye(128, dtype=jnp.float32)
    out_t = pl.pallas_call(
        _tc_body,
        grid=(N_TOKEN,),
        in_specs=[
            pl.BlockSpec((bb, 128), lambda t: (t, 0)),
            pl.BlockSpec((1, 8, bb), lambda t: (t, 0, 0)),
            pl.BlockSpec((1, N_EMBED, 128), lambda t: (t, 0, 0)),
            pl.BlockSpec((128, 128), lambda t: (0, 0)),
        ],
        out_specs=pl.BlockSpec((1, N_EMBED, bb), lambda t: (t, 0, 0)),
        out_shape=jax.ShapeDtypeStruct((N_TOKEN, N_EMBED, BATCH), jnp.float32),
    )(inter, par8, pos_x, eye)
    return out_t


def kernel(tokens, token_embedding, position_embedding):
    tokens_t = jnp.asarray(tokens, jnp.int32).T
    table_t = token_embedding.T
    pos_x = jnp.broadcast_to(
        position_embedding[:, :, None], (N_TOKEN, N_EMBED, 128))
    out_t = _run(tokens_t, table_t, pos_x)
    return jnp.transpose(out_t, (2, 0, 1))


# speedup vs baseline: 2.1577x; 1.0845x over previous
import functools

import jax
import jax.numpy as jnp
from jax import lax
from jax.experimental import pallas as pl
from jax.experimental.pallas import tpu as pltpu
from jax.experimental.pallas import tpu_sc as plsc

N_VOCAB = 1000000
N_EMBED = 64
N_TOKEN = 50
BATCH = 4096

ROWS = BATCH * N_TOKEN
NC, NS = 2, 16
NW = NC * NS
BPW = BATCH // NW
LANES = 16

_GB = 16384
_GBLOG = 14
_HB = _GB // 2
_HBLOG = 13


def _sc_body(tok_hbm, tab_hbm, inter_hbm, par_hbm,
             idx_v, pidx_v, par_v, buf_v, gsem0, gsem1, ssem0, ssem1, psem):
    wid = lax.axis_index("s") * NC + lax.axis_index("c")
    b0 = wid * BPW

    pltpu.sync_copy(tok_hbm.at[:, pl.ds(b0, BPW)], idx_v)

    def prep_row(t, carry):
        for j in range(BPW // LANES):
            sl = pl.ds(j * LANES, LANES)
            v = idx_v[t, sl]
            pidx_v[t, sl] = lax.bitwise_or(
                lax.shift_left(lax.shift_right_logical(v, _GBLOG), _HBLOG),
                lax.bitwise_and(v, _HB - 1))
            p = lax.bitwise_and(lax.shift_right_logical(v, _HBLOG), 1)
            for r in range(8):
                par_v[t, r, sl] = p
        return carry

    lax.fori_loop(0, N_TOKEN, prep_row, 0)
    par_handle = pltpu.async_copy(
        par_v, par_hbm.at[:, :, pl.ds(b0, BPW)], psem)

    gsems = (gsem0, gsem1)
    ssems = (ssem0, ssem1)
    gather_handles = [None, None]
    store_handles = [None, None]

    def start_gather(t, b):
        if store_handles[b] is not None:
            store_handles[b].wait()
            store_handles[b] = None
        gather_handles[b] = pltpu.async_copy(
            tab_hbm.at[pidx_v.at[t]], buf_v.at[b], gsems[b])

    start_gather(0, 0)
    for t in range(N_TOKEN):
        b = t % 2
        if t + 1 < N_TOKEN:
            start_gather(t + 1, 1 - b)
        gather_handles[b].wait()
        store_handles[b] = pltpu.async_copy(
            buf_v.at[b],
            inter_hbm.at[pl.ds(t * BATCH + b0, BPW)],
            ssems[b])
    for b in range(2):
        if store_handles[b] is not None:
            store_handles[b].wait()
    par_handle.wait()


_NPAIR_BLOCKS = (N_VOCAB + _GB - 1) // _GB
_NPAIR = _NPAIR_BLOCKS * (_GB // 2)


def _pack_body(tp_ref, o_ref):
    x = tp_ref[...]
    y = jnp.transpose(x)
    o_ref[...] = jnp.concatenate([y[: _GB // 2], y[_GB // 2:]], axis=1)


def _pack_table(table_t):
    return pl.pallas_call(
        _pack_body,
        grid=(_NPAIR_BLOCKS,),
        in_specs=[pl.BlockSpec((N_EMBED, _GB), lambda j: (0, j))],
        out_specs=pl.BlockSpec((_GB // 2, 128), lambda j: (j, 0)),
        out_shape=jax.ShapeDtypeStruct((_NPAIR, 128), jnp.float32),
    )(table_t)


@jax.jit
def _run(tokens_t, table_t, pos_x):
    table_pairs = _pack_table(table_t)
    sc_gather = functools.partial(
        pl.kernel,
        mesh=plsc.VectorSubcoreMesh(core_axis_name="c", subcore_axis_name="s"),
        compiler_params=pltpu.CompilerParams(use_tc_tiling_on_sc=True),
        out_type=(
            jax.ShapeDtypeStruct((ROWS, 128), jnp.float32),
            jax.ShapeDtypeStruct((N_TOKEN, 8, BATCH), jnp.int32),
        ),
        scratch_types=[
            pltpu.VMEM((N_TOKEN, BPW), jnp.int32),
            pltpu.VMEM((N_TOKEN, BPW), jnp.int32),
            pltpu.VMEM((N_TOKEN, 8, BPW), jnp.int32),
            pltpu.VMEM((2, BPW, 128), jnp.float32),
            pltpu.SemaphoreType.DMA,
            pltpu.SemaphoreType.DMA,
            pltpu.SemaphoreType.DMA,
            pltpu.SemaphoreType.DMA,
            pltpu.SemaphoreType.DMA,
        ],
    )
    inter, par8 = sc_gather(_sc_body)(tokens_t, table_pairs)

    bb = BATCH

    def _tc_body(x_ref, par_ref, pos_ref, eye_ref, o_ref):
        xt = lax.dot_general(eye_ref[...], x_ref[...],
                             (((1,), (1,)), ((), ())),
                             preferred_element_type=jnp.float32)
        lo = xt[:N_EMBED, :]
        hi = xt[N_EMBED:, :]
        par = par_ref[0, 0:1, :]
        sel = jnp.where(par == 1, hi, lo)
        posv = pos_ref[0][:, 0:1]
        o_ref[0] = sel + posv

    eye = jnp.eye(128, dtype=jnp.float32)
    out_t = pl.pallas_call(
        _tc_body,
        grid=(N_TOKEN,),
        in_specs=[
            pl.BlockSpec((bb, 128), lambda t: (t, 0)),
            pl.BlockSpec((1, 8, bb), lambda t: (t, 0, 0)),
            pl.BlockSpec((1, N_EMBED, 128), lambda t: (t, 0, 0)),
            pl.BlockSpec((128, 128), lambda t: (0, 0)),
        ],
        out_specs=pl.BlockSpec((1, N_EMBED, bb), lambda t: (t, 0, 0)),
        out_shape=jax.ShapeDtypeStruct((N_TOKEN, N_EMBED, BATCH), jnp.float32),
    )(inter, par8, pos_x, eye)
    return out_t


def kernel(tokens, token_embedding, position_embedding):
    tokens_t = jnp.asarray(tokens, jnp.int32).T
    table_t = token_embedding.T
    pos_x = jnp.broadcast_to(
        position_embedding[:, :, None], (N_TOKEN, N_EMBED, 128))
    out_t = _run(tokens_t, table_t, pos_x)
    return jnp.transpose(out_t, (2, 0, 1))
